# baseline (speedup 1.0000x reference)
import functools

import jax
import jax.numpy as jnp
from jax import lax
from jax.experimental import pallas as pl
from jax.experimental.pallas import tpu as pltpu
from jax.experimental.pallas import tpu_sc as plsc

TOK = 4096
EMBED = 1024
HE = 1024
E = 8
NMOE = 2048
GL_W = 128
BLK = 128
NSLOT = NMOE + E * BLK
NBLK = NSLOT // BLK
TB = 512
SB = 512
NC, NS = 2, 16
NW = NC * NS
F32 = jnp.float32
I32 = jnp.int32
HI = lax.Precision.HIGHEST


def _iotaf(shape, dim):
    return lax.broadcasted_iota(I32, shape, dim).astype(F32)


def _mm_t(a, b):
    return lax.dot_general(a, b, (((1,), (1,)), ((), ())),
                           preferred_element_type=F32)


def _swiglu_ln(xb, w1, b1, w2, b2, g, bb):
    x1 = _mm_t(xb, w1) + b1
    x2 = _mm_t(xb, w2) + b2
    hh = x1 * jax.nn.sigmoid(x1) * x2
    m = jnp.mean(hh, axis=-1, keepdims=True)
    v = jnp.mean((hh - m) ** 2, axis=-1, keepdims=True)
    return (hh - m) / jnp.sqrt(v + 1e-5) * g + bb


def _k1g_body(x_ref, gw_ref, gb_ref, gl_ref):
    gl_ref[...] = _mm_t(x_ref[...], gw_ref[...]) + gb_ref[...][None, :]


def _k1g(x, gate_w, gate_b):
    n = TOK // TB
    full2 = lambda shp: pl.BlockSpec(shp, lambda i: (0,) * len(shp))
    return pl.pallas_call(
        _k1g_body,
        grid=(n,),
        in_specs=[pl.BlockSpec((TB, EMBED), lambda i: (i, 0)),
                  full2((GL_W, EMBED)), full2((GL_W,))],
        out_specs=pl.BlockSpec((TB, GL_W), lambda i: (i, 0)),
        out_shape=jax.ShapeDtypeStruct((TOK, GL_W), F32),
    )(x, gate_w, gate_b)


def _k1s_body(x_ref, w1_ref, b1_ref, w2_ref, b2_ref, g_ref, bb_ref,
              w3_ref, b3_ref, so_ref):
    h = _swiglu_ln(x_ref[...], w1_ref[...], b1_ref[...][None, :],
                   w2_ref[...], b2_ref[...][None, :], g_ref[...][None, :],
                   bb_ref[...][None, :])
    so_ref[...] = _mm_t(h, w3_ref[...]) + b3_ref[...][None, :]


def _k1s(x, sw1_w, sw1_b, sw2_w, sw2_b, sln_g, sln_b, sw3_w, sw3_b):
    n = TOK // TB
    full2 = lambda shp: pl.BlockSpec(shp, lambda i: (0,) * len(shp))
    return pl.pallas_call(
        _k1s_body,
        grid=(n,),
        in_specs=[
            pl.BlockSpec((TB, EMBED), lambda i: (i, 0)),
            full2((HE, EMBED)), full2((HE,)),
            full2((HE, EMBED)), full2((HE,)),
            full2((HE,)), full2((HE,)),
            full2((EMBED, HE)), full2((EMBED,)),
        ],
        out_specs=pl.BlockSpec((TB, EMBED), lambda i: (i, 0)),
        out_shape=jax.ShapeDtypeStruct((TOK, EMBED), F32),
    )(x, sw1_w, sw1_b, sw2_w, sw2_b, sln_g, sln_b, sw3_w, sw3_b)


def _k2a_body(gl_ref, idxc_ref, idxr_ref, eb_ref,
              destrow_ref, p_ref, meta_ref):
    gl = gl_ref[...]
    idx_col = idxc_ref[...]
    idx_row = idxr_ref[...]
    rows = _iotaf((NMOE, NMOE), 0)
    cols = _iotaf((NMOE, NMOE), 1)
    occb = jnp.sum(((cols < rows) & (idx_col == idx_row)).astype(F32),
                   axis=1, keepdims=True)
    firstf = (occb == 0).astype(F32)

    ss = jax.nn.sigmoid(gl[:, 0:1])
    es = jax.nn.sigmoid(gl[:, 1:1 + E])
    sel = es + eb_ref[...]
    best_sel = sel[:, 0:1]
    best_es = es[:, 0:1]
    top1 = jnp.zeros_like(ss)
    for e in range(1, E):
        take = sel[:, e:e + 1] > best_sel
        best_sel = jnp.where(take, sel[:, e:e + 1], best_sel)
        best_es = jnp.where(take, es[:, e:e + 1], best_es)
        top1 = jnp.where(take, float(e), top1)
    wsh = jax.nn.sigmoid(ss - best_es)
    wex = 1.0 - wsh

    erow = _iotaf((NMOE, E), 1)
    oh_raw = (top1 == erow).astype(F32)
    ohf = oh_raw * firstf
    counts = jnp.sum(ohf, axis=0, keepdims=True)
    pc = jnp.floor((counts + (BLK - 1)) / BLK) * BLK
    t8r = _iotaf((E, E), 0)
    t8c = _iotaf((E, E), 1)
    poff = jnp.dot(pc, (t8r < t8c).astype(F32),
                   preferred_element_type=F32, precision=HI)
    tril = (cols < rows).astype(F32)
    cum = jnp.dot(tril, ohf, preferred_element_type=F32,
                  precision=HI)
    rank = jnp.sum(cum * oh_raw, axis=1, keepdims=True)
    dest = jnp.sum(oh_raw * poff, axis=1, keepdims=True) + rank
    dest = jnp.where(firstf > 0, dest, -1.0)

    ncol = _iotaf((NMOE, E), 0)
    firstn = jnp.min(jnp.where(ohf > 0, ncol, float(NMOE)),
                     axis=0, keepdims=True)
    erow1 = _iotaf((1, E), 1)
    e_first = jnp.min(jnp.where(counts > 0, erow1, float(E)),
                      axis=1, keepdims=True)
    e_last = jnp.max(jnp.where(counts > 0, erow1, -1.0),
                     axis=1, keepdims=True)
    sum_pc = jnp.sum(pc, axis=1, keepdims=True)
    nab = sum_pc / BLK
    sblk_col = _iotaf((NBLK, 1), 0) * BLK
    cond = (poff <= sblk_col) & (pc > 0)
    eids = _iotaf((NBLK, E), 1)
    be_col = jnp.max(jnp.where(cond, eids, -1.0), axis=1, keepdims=True)
    be_col = jnp.where(sblk_col >= sum_pc, e_last, be_col)

    p_ref[...] = jnp.concatenate(
        [jnp.where(firstf > 0, dest, 0.0), wsh * firstf, wex * firstf,
         firstf], axis=1)
    pk = jnp.concatenate(
        [dest, jnp.concatenate([be_col,
                                jnp.zeros((NMOE - NBLK, 1), F32)], 0)],
        axis=1)
    pkt = pk.T
    destrow_ref[...] = pkt[0:1, :]
    be_row = pkt[1:2, 0:NBLK]
    meta = jnp.concatenate(
        [counts, pc, poff, firstn, be_row, e_first, sum_pc, nab, e_last,
         jnp.zeros((1, 128 - 60), F32)], axis=1)
    meta_ref[...] = meta.astype(I32)


def _k2a(gl_n, idx_col, idx_row, eb_row):
    full2 = lambda shp: pl.BlockSpec(shp, lambda: (0,) * len(shp))
    return pl.pallas_call(
        _k2a_body,
        in_specs=[full2((NMOE, GL_W)), full2((NMOE, 1)), full2((1, NMOE)),
                  full2((1, E))],
        out_specs=[full2((1, NMOE)), full2((NMOE, 4)), full2((1, 128))],
        out_shape=[jax.ShapeDtypeStruct((1, NMOE), F32),
                   jax.ShapeDtypeStruct((NMOE, 4), F32),
                   jax.ShapeDtypeStruct((1, 128), I32)],
        compiler_params=pltpu.CompilerParams(
            vmem_limit_bytes=100 * 1024 * 1024),
    )(gl_n, idx_col, idx_row, eb_row)


def _k2b_body(dest_ref, p_ref, meta_ref, src_ref, ew_ref):
    b = pl.program_id(0)
    base = (b * SB).astype(F32)
    s_col = _iotaf((SB, 1), 0) + base
    m = (s_col == dest_ref[...]).astype(F32)
    iot_n = _iotaf((NMOE, 2), 0)
    onz = _iotaf((NMOE, 2), 1)
    p2 = jnp.where(onz > 0, 1.0, iot_n)
    p3 = jnp.concatenate([p2, p_ref[:, 2:3]], axis=1)
    r = jnp.dot(m, p3, preferred_element_type=F32, precision=HI)
    src_hit = r[:, 0:1]
    hit = r[:, 1:2]
    ew_ref[...] = r[:, 2:3]
    e_slot = jnp.full((SB, 1), -1.0, F32)
    for e in range(E):
        poff_e = meta_ref[0, 16 + e].astype(F32)
        cond = (s_col >= poff_e) & (meta_ref[0, 8 + e] > 0)
        e_slot = jnp.where(cond, float(e), e_slot)
    sum_pc = meta_ref[0, 57].astype(F32)
    e_lastf = meta_ref[0, 59].astype(F32)
    e_slot = jnp.where(s_col >= sum_pc, e_lastf, e_slot)
    fill = jnp.zeros((SB, 1), F32)
    for e in range(E):
        fill = jnp.where(e_slot == float(e),
                         meta_ref[0, 24 + e].astype(F32), fill)
    src_ref[...] = jnp.where(hit > 0.5, src_hit, fill).astype(I32)


def _k2b(dest_row, p, meta):
    return pl.pallas_call(
        _k2b_body,
        grid=(NSLOT // SB,),
        in_specs=[pl.BlockSpec((1, NMOE), lambda b: (0, 0)),
                  pl.BlockSpec((NMOE, 4), lambda b: (0, 0)),
                  pl.BlockSpec(memory_space=pltpu.SMEM)],
        out_specs=[pl.BlockSpec((SB, 1), lambda b: (b, 0)),
                   pl.BlockSpec((SB, 1), lambda b: (b, 0))],
        out_shape=[jax.ShapeDtypeStruct((NSLOT, 1), I32),
                   jax.ShapeDtypeStruct((NSLOT, 1), F32)],
    )(dest_row, p, meta)


def _k2c_body(idxr_ref, p_ref, sh_ref, ss_ref, s_ref):
    b = pl.program_id(0)
    r_col = _iotaf((SB, 1), 0) + (b * SB).astype(F32)
    m = (r_col == idxr_ref[...]).astype(F32)
    r4 = jnp.dot(m, p_ref[...], preferred_element_type=F32, precision=HI)
    routed = r4[:, 3:4] > 0.5
    a_col = jnp.where(routed, r4[:, 1:2], 1.0)
    ss_ref[...] = sh_ref[...] * a_col
    slot = jnp.where(routed, r4[:, 0:1], float(NSLOT - 1))
    seg = EMBED // 128
    s_ref[...] = (slot * seg + _iotaf((SB, seg), 1)).astype(I32)


def _k2c(idx_row, p, shared_out):
    n = TOK // SB
    return pl.pallas_call(
        _k2c_body,
        grid=(n,),
        in_specs=[pl.BlockSpec((1, NMOE), lambda b: (0, 0)),
                  pl.BlockSpec((NMOE, 4), lambda b: (0, 0)),
                  pl.BlockSpec((SB, EMBED), lambda b: (b, 0))],
        out_specs=[pl.BlockSpec((SB, EMBED), lambda b: (b, 0)),
                   pl.BlockSpec((SB, EMBED // 128), lambda b: (b, 0))],
        out_shape=[jax.ShapeDtypeStruct((TOK, EMBED), F32),
                   jax.ShapeDtypeStruct((TOK, EMBED // 128), I32)],
    )(idx_row, p, shared_out)


def _k3_body(bexp_ref, nab_ref, x_ref, w1_ref, b1_ref, w2_ref, b2_ref,
             g_ref, bb_ref, w3_ref, b3_ref, ew_ref, y_ref):
    b = pl.program_id(0)

    @pl.when(b < nab_ref[0])
    def _():
        h = _swiglu_ln(x_ref[...], w1_ref[0], b1_ref[0], w2_ref[0],
                       b2_ref[0], g_ref[...][None, :],
                       bb_ref[...][None, :])
        y_ref[...] = (_mm_t(h, w3_ref[0]) + b3_ref[0]) * ew_ref[...]

    @pl.when(b >= nab_ref[0])
    def _():
        y_ref[...] = jnp.zeros_like(y_ref)


def _k3(bexp, nabv, x_sorted, w1, b1, w2, b2, seln_g, seln_b, w3, b3,
        ew_sorted):
    grid_spec = pltpu.PrefetchScalarGridSpec(
        num_scalar_prefetch=2,
        grid=(NBLK,),
        in_specs=[
            pl.BlockSpec((BLK, EMBED), lambda b, be, na: (b, 0)),
            pl.BlockSpec((1, HE, EMBED), lambda b, be, na: (be[b], 0, 0)),
            pl.BlockSpec((1, 1, HE), lambda b, be, na: (be[b], 0, 0)),
            pl.BlockSpec((1, HE, EMBED), lambda b, be, na: (be[b], 0, 0)),
            pl.BlockSpec((1, 1, HE), lambda b, be, na: (be[b], 0, 0)),
            pl.BlockSpec((HE,), lambda b, be, na: (0,)),
            pl.BlockSpec((HE,), lambda b, be, na: (0,)),
            pl.BlockSpec((1, EMBED, HE), lambda b, be, na: (be[b], 0, 0)),
            pl.BlockSpec((1, 1, EMBED), lambda b, be, na: (be[b], 0, 0)),
            pl.BlockSpec((BLK, 1), lambda b, be, na: (b, 0)),
        ],
        out_specs=pl.BlockSpec((BLK, EMBED), lambda b, be, na: (b, 0)),
    )
    return pl.pallas_call(
        _k3_body,
        grid_spec=grid_spec,
        out_shape=jax.ShapeDtypeStruct((NSLOT, EMBED), F32),
        compiler_params=pltpu.CompilerParams(
            vmem_limit_bytes=100 * 1024 * 1024),
    )(bexp, nabv, x_sorted, w1, b1, w2, b2, seln_g, seln_b, w3, b3,
      ew_sorted)


def _sc_mesh():
    return plsc.VectorSubcoreMesh(core_axis_name="c", subcore_axis_name="s",
                                  num_cores=NC, num_subcores=NS)


def _wid():
    return lax.axis_index("s") * NC + lax.axis_index("c")


def _s1(gl_all, index_i):
    bpw = NMOE // NW

    @functools.partial(
        pl.kernel, mesh=_sc_mesh(),
        out_type=jax.ShapeDtypeStruct((NMOE, GL_W), F32),
        scratch_types=[pltpu.VMEM((bpw,), I32),
                       pltpu.VMEM((bpw, GL_W), F32),
                       pltpu.SemaphoreType.DMA])
    def k(gl_hbm, idx_hbm, out_hbm, idx_v, rows_v, sem):
        base = _wid() * bpw
        pltpu.sync_copy(idx_hbm.at[pl.ds(base, bpw)], idx_v)
        pltpu.async_copy(gl_hbm.at[idx_v], rows_v, sem).wait()
        pltpu.sync_copy(rows_v, out_hbm.at[pl.ds(base, bpw)])

    return k(gl_all, index_i)


def _s2(x, index_i, sorted_src):
    spw = NSLOT // NW
    ch = 16
    nb = 3
    gs = spw // nb

    @functools.partial(
        pl.kernel, mesh=_sc_mesh(),
        out_type=jax.ShapeDtypeStruct((NSLOT, EMBED), F32),
        compiler_params=pltpu.CompilerParams(needs_layout_passes=False),
        scratch_types=[pltpu.VMEM((NMOE,), I32),
                       pltpu.VMEM((spw,), I32),
                       pltpu.VMEM((spw,), I32)]
        + [pltpu.VMEM((gs, EMBED), F32)] * 3
        + [pltpu.SemaphoreType.DMA] * 6)
    def k(x_hbm, idx_hbm, src_hbm, out_hbm, tab_v, src_v, tid_v,
          b0, b1, b2, sg0, sg1, sg2, so0, so1, so2):
        base = _wid() * spw
        bufs = (b0, b1, b2)
        sg = (sg0, sg1, sg2)
        so = (so0, so1, so2)
        pltpu.sync_copy(idx_hbm, tab_v)
        pltpu.sync_copy(src_hbm.at[pl.ds(base, spw)], src_v)
        for c in range(spw // ch):
            sv = src_v[pl.ds(c * ch, ch)]
            tid_v[pl.ds(c * ch, ch)] = plsc.load_gather(tab_v, [sv])
        gh = [None] * nb
        oh = [None] * nb
        for s in range(nb):
            gh[s] = pltpu.async_copy(
                x_hbm.at[tid_v.at[pl.ds(s * gs, gs)]], bufs[s], sg[s])
        for s in range(nb):
            gh[s].wait()
            oh[s] = pltpu.async_copy(
                bufs[s], out_hbm.at[pl.ds(base + s * gs, gs)], so[s])
        for s in range(nb):
            oh[s].wait()

    return k(x, index_i, sorted_src)


def _s3(sscaled, y_scaled, fslot):
    rpw = TOK // NW
    ch = 16
    seg = EMBED // 128
    nb = 4
    nch = rpw // ch

    ss_f = sscaled.reshape(TOK * seg, 128)
    y_f = y_scaled.reshape(NSLOT * seg, 128)
    fs_f = fslot.reshape(TOK * seg)

    @functools.partial(
        pl.kernel, mesh=_sc_mesh(),
        out_type=jax.ShapeDtypeStruct((TOK * seg, 128), F32),
        scratch_types=[pltpu.VMEM((ch * seg,), I32)] * nb
        + [pltpu.VMEM((ch * seg, 128), F32)] * nb
        + [pltpu.SemaphoreType.DMA] * (3 * nb))
    def k(ss_hbm, y_hbm, fs_hbm, out_hbm, *scr):
        sbs = scr[0:nb]
        bufs = scr[nb:2 * nb]
        sin = scr[2 * nb:3 * nb]
        sfs = scr[3 * nb:4 * nb]
        sout = scr[4 * nb:5 * nb]
        base = _wid() * rpw
        ih = [None] * nb
        fh = [None] * nb
        oh = [None] * nb

        def issue(c):
            s = c % nb
            if oh[s] is not None:
                oh[s].wait()
                oh[s] = None
            fbase = (base + c * ch) * seg
            ih[s] = pltpu.async_copy(ss_hbm.at[pl.ds(fbase, ch * seg)],
                                     bufs[s], sin[s])
            fh[s] = pltpu.async_copy(fs_hbm.at[pl.ds(fbase, ch * seg)],
                                     sbs[s], sfs[s])

        for c in range(min(nb, nch)):
            issue(c)
        for c in range(nch):
            s = c % nb
            fbase = (base + c * ch) * seg
            ih[s].wait()
            fh[s].wait()
            pltpu.async_copy(y_hbm.at[sbs[s]], bufs[s], sin[s],
                             add=True).wait()
            oh[s] = pltpu.async_copy(
                bufs[s], out_hbm.at[pl.ds(fbase, ch * seg)], sout[s])
            if c + nb < nch:
                issue(c + nb)
        for s in range(nb):
            if oh[s] is not None:
                oh[s].wait()

    return k(ss_f, y_f, fs_f).reshape(TOK, EMBED)


def kernel(x, index, sw1_w, sw1_b, sw2_w, sw2_b, sln_g, sln_b, sw3_w,
           sw3_b, sg_w, sg_b, eg_w, eg_b, w1, b1, w2, b2, seln_g, seln_b,
           w3, b3, exp_bias):
    index_i = index.astype(I32)
    idx_f = index_i.astype(F32)
    idx_col = idx_f.reshape(NMOE, 1)
    idx_row = idx_f.reshape(1, NMOE)
    eb_row = exp_bias.reshape(1, E).astype(F32)
    gate_w = jnp.zeros((GL_W, EMBED), F32).at[0:1].set(sg_w)
    gate_w = gate_w.at[1:1 + E].set(eg_w)
    gate_b = jnp.zeros((GL_W,), F32).at[0:1].set(sg_b)
    gate_b = gate_b.at[1:1 + E].set(eg_b)

    gl_all = _k1g(x, gate_w, gate_b)
    gl_n = _s1(gl_all, index_i)
    dest_row, p, meta = _k2a(gl_n, idx_col, idx_row, eb_row)
    meta_flat = meta.reshape(128)
    bexp = meta_flat[32:56]
    nabv = meta_flat[58:59]
    sorted_src, ew_sorted = _k2b(dest_row, p, meta)
    x_sorted = _s2(x, index_i, sorted_src.reshape(NSLOT))
    shared_out = _k1s(x, sw1_w, sw1_b, sw2_w, sw2_b, sln_g, sln_b,
                      sw3_w, sw3_b)
    y_scaled = _k3(bexp, nabv, x_sorted, w1, b1.reshape(E, 1, HE), w2,
                   b2.reshape(E, 1, HE), seln_g, seln_b, w3,
                   b3.reshape(E, 1, EMBED), ew_sorted)
    sscaled, fslot = _k2c(idx_row, p, shared_out)
    out = _s3(sscaled, y_scaled, fslot)
    return out


# speedup vs baseline: 1.2742x; 1.2742x over previous
import functools

import jax
import jax.numpy as jnp
from jax import lax
from jax.experimental import pallas as pl
from jax.experimental.pallas import tpu as pltpu
from jax.experimental.pallas import tpu_sc as plsc

TOK = 4096
EMBED = 1024
HE = 1024
E = 8
NMOE = 2048
GL_W = 128
BLK = 128
NSLOT = NMOE + E * BLK
NBLK = NSLOT // BLK
TB = 512
SB = 512
NC, NS = 2, 16
NW = NC * NS
F32 = jnp.float32
I32 = jnp.int32
HI = lax.Precision.HIGHEST


def _iotaf(shape, dim):
    return lax.broadcasted_iota(I32, shape, dim).astype(F32)


def _mm_t(a, b):
    return lax.dot_general(a, b, (((1,), (1,)), ((), ())),
                           preferred_element_type=F32)


def _swiglu_ln(xb, w1, b1, w2, b2, g, bb):
    x1 = _mm_t(xb, w1) + b1
    x2 = _mm_t(xb, w2) + b2
    hh = x1 * jax.nn.sigmoid(x1) * x2
    m = jnp.mean(hh, axis=-1, keepdims=True)
    v = jnp.mean((hh - m) ** 2, axis=-1, keepdims=True)
    return (hh - m) / jnp.sqrt(v + 1e-5) * g + bb


def _k1g_body(x_ref, gw_ref, gb_ref, gl_ref):
    gl_ref[...] = _mm_t(x_ref[...], gw_ref[...]) + gb_ref[...][None, :]


def _k1g(x, gate_w, gate_b):
    n = TOK // TB
    full2 = lambda shp: pl.BlockSpec(shp, lambda i: (0,) * len(shp))
    return pl.pallas_call(
        _k1g_body,
        grid=(n,),
        in_specs=[pl.BlockSpec((TB, EMBED), lambda i: (i, 0)),
                  full2((GL_W, EMBED)), full2((GL_W,))],
        out_specs=pl.BlockSpec((TB, GL_W), lambda i: (i, 0)),
        out_shape=jax.ShapeDtypeStruct((TOK, GL_W), F32),
    )(x, gate_w, gate_b)


def _k1s_body(x_ref, w1_ref, b1_ref, w2_ref, b2_ref, g_ref, bb_ref,
              w3_ref, b3_ref, so_ref):
    h = _swiglu_ln(x_ref[...], w1_ref[...], b1_ref[...][None, :],
                   w2_ref[...], b2_ref[...][None, :], g_ref[...][None, :],
                   bb_ref[...][None, :])
    so_ref[...] = _mm_t(h, w3_ref[...]) + b3_ref[...][None, :]


def _k1s(x, sw1_w, sw1_b, sw2_w, sw2_b, sln_g, sln_b, sw3_w, sw3_b):
    n = TOK // TB
    full2 = lambda shp: pl.BlockSpec(shp, lambda i: (0,) * len(shp))
    return pl.pallas_call(
        _k1s_body,
        grid=(n,),
        in_specs=[
            pl.BlockSpec((TB, EMBED), lambda i: (i, 0)),
            full2((HE, EMBED)), full2((HE,)),
            full2((HE, EMBED)), full2((HE,)),
            full2((HE,)), full2((HE,)),
            full2((EMBED, HE)), full2((EMBED,)),
        ],
        out_specs=pl.BlockSpec((TB, EMBED), lambda i: (i, 0)),
        out_shape=jax.ShapeDtypeStruct((TOK, EMBED), F32),
    )(x, sw1_w, sw1_b, sw2_w, sw2_b, sln_g, sln_b, sw3_w, sw3_b)


def _k2a_body(gl_ref, idxc_ref, idxr_ref, eb_ref,
              destrow_ref, p_ref, meta_ref):
    gl = gl_ref[...]
    idx_col = idxc_ref[...]
    idx_row = idxr_ref[...]
    rows = _iotaf((NMOE, NMOE), 0)
    cols = _iotaf((NMOE, NMOE), 1)
    occb = jnp.sum(((cols < rows) & (idx_col == idx_row)).astype(F32),
                   axis=1, keepdims=True)
    firstf = (occb == 0).astype(F32)

    ss = jax.nn.sigmoid(gl[:, 0:1])
    es = jax.nn.sigmoid(gl[:, 1:1 + E])
    sel = es + eb_ref[...]
    best_sel = sel[:, 0:1]
    best_es = es[:, 0:1]
    top1 = jnp.zeros_like(ss)
    for e in range(1, E):
        take = sel[:, e:e + 1] > best_sel
        best_sel = jnp.where(take, sel[:, e:e + 1], best_sel)
        best_es = jnp.where(take, es[:, e:e + 1], best_es)
        top1 = jnp.where(take, float(e), top1)
    wsh = jax.nn.sigmoid(ss - best_es)
    wex = 1.0 - wsh

    erow = _iotaf((NMOE, E), 1)
    oh_raw = (top1 == erow).astype(F32)
    ohf = oh_raw * firstf
    counts = jnp.sum(ohf, axis=0, keepdims=True)
    pc = jnp.floor((counts + (BLK - 1)) / BLK) * BLK
    t8r = _iotaf((E, E), 0)
    t8c = _iotaf((E, E), 1)
    poff = jnp.dot(pc, (t8r < t8c).astype(F32),
                   preferred_element_type=F32, precision=HI)
    tril = (cols < rows).astype(F32)
    cum = jnp.dot(tril, ohf, preferred_element_type=F32,
                  precision=HI)
    rank = jnp.sum(cum * oh_raw, axis=1, keepdims=True)
    dest = jnp.sum(oh_raw * poff, axis=1, keepdims=True) + rank
    dest = jnp.where(firstf > 0, dest, -1.0)

    ncol = _iotaf((NMOE, E), 0)
    firstn = jnp.min(jnp.where(ohf > 0, ncol, float(NMOE)),
                     axis=0, keepdims=True)
    erow1 = _iotaf((1, E), 1)
    e_first = jnp.min(jnp.where(counts > 0, erow1, float(E)),
                      axis=1, keepdims=True)
    e_last = jnp.max(jnp.where(counts > 0, erow1, -1.0),
                     axis=1, keepdims=True)
    sum_pc = jnp.sum(pc, axis=1, keepdims=True)
    nab = sum_pc / BLK
    sblk_col = _iotaf((NBLK, 1), 0) * BLK
    cond = (poff <= sblk_col) & (pc > 0)
    eids = _iotaf((NBLK, E), 1)
    be_col = jnp.max(jnp.where(cond, eids, -1.0), axis=1, keepdims=True)
    be_col = jnp.where(sblk_col >= sum_pc, e_last, be_col)

    p_ref[...] = jnp.concatenate(
        [jnp.where(firstf > 0, dest, 0.0), wsh * firstf, wex * firstf,
         firstf], axis=1)
    pk = jnp.concatenate(
        [dest, jnp.concatenate([be_col,
                                jnp.zeros((NMOE - NBLK, 1), F32)], 0)],
        axis=1)
    pkt = pk.T
    destrow_ref[...] = pkt[0:1, :]
    be_row = pkt[1:2, 0:NBLK]
    meta = jnp.concatenate(
        [counts, pc, poff, firstn, be_row, e_first, sum_pc, nab, e_last,
         jnp.zeros((1, 128 - 60), F32)], axis=1)
    meta_ref[...] = meta.astype(I32)


def _k2a(gl_n, idx_col, idx_row, eb_row):
    full2 = lambda shp: pl.BlockSpec(shp, lambda: (0,) * len(shp))
    return pl.pallas_call(
        _k2a_body,
        in_specs=[full2((NMOE, GL_W)), full2((NMOE, 1)), full2((1, NMOE)),
                  full2((1, E))],
        out_specs=[full2((1, NMOE)), full2((NMOE, 4)), full2((1, 128))],
        out_shape=[jax.ShapeDtypeStruct((1, NMOE), F32),
                   jax.ShapeDtypeStruct((NMOE, 4), F32),
                   jax.ShapeDtypeStruct((1, 128), I32)],
        compiler_params=pltpu.CompilerParams(
            vmem_limit_bytes=100 * 1024 * 1024),
    )(gl_n, idx_col, idx_row, eb_row)


def _k2b_body(dest_ref, p_ref, meta_ref, src_ref, ew_ref):
    b = pl.program_id(0)
    base = (b * SB).astype(F32)
    s_col = _iotaf((SB, 1), 0) + base
    m = (s_col == dest_ref[...]).astype(F32)
    iot_n = _iotaf((NMOE, 2), 0)
    onz = _iotaf((NMOE, 2), 1)
    p2 = jnp.where(onz > 0, 1.0, iot_n)
    p3 = jnp.concatenate([p2, p_ref[:, 2:3]], axis=1)
    r = jnp.dot(m, p3, preferred_element_type=F32, precision=HI)
    src_hit = r[:, 0:1]
    hit = r[:, 1:2]
    ew_ref[...] = r[:, 2:3]
    e_slot = jnp.full((SB, 1), -1.0, F32)
    for e in range(E):
        poff_e = meta_ref[0, 16 + e].astype(F32)
        cond = (s_col >= poff_e) & (meta_ref[0, 8 + e] > 0)
        e_slot = jnp.where(cond, float(e), e_slot)
    sum_pc = meta_ref[0, 57].astype(F32)
    e_lastf = meta_ref[0, 59].astype(F32)
    e_slot = jnp.where(s_col >= sum_pc, e_lastf, e_slot)
    fill = jnp.zeros((SB, 1), F32)
    for e in range(E):
        fill = jnp.where(e_slot == float(e),
                         meta_ref[0, 24 + e].astype(F32), fill)
    src_ref[...] = jnp.where(hit > 0.5, src_hit, fill).astype(I32)


def _k2b(dest_row, p, meta):
    return pl.pallas_call(
        _k2b_body,
        grid=(NSLOT // SB,),
        in_specs=[pl.BlockSpec((1, NMOE), lambda b: (0, 0)),
                  pl.BlockSpec((NMOE, 4), lambda b: (0, 0)),
                  pl.BlockSpec(memory_space=pltpu.SMEM)],
        out_specs=[pl.BlockSpec((SB, 1), lambda b: (b, 0)),
                   pl.BlockSpec((SB, 1), lambda b: (b, 0))],
        out_shape=[jax.ShapeDtypeStruct((NSLOT, 1), I32),
                   jax.ShapeDtypeStruct((NSLOT, 1), F32)],
    )(dest_row, p, meta)


def _k2c_body(idxr_ref, p_ref, sh_ref, y_ref, out_ref):
    b = pl.program_id(0)
    r_col = _iotaf((SB, 1), 0) + (b * SB).astype(F32)
    m = (r_col == idxr_ref[...]).astype(F32)
    r4 = jnp.dot(m, p_ref[...], preferred_element_type=F32, precision=HI)
    routed = r4[:, 3:4] > 0.5
    a_col = jnp.where(routed, r4[:, 1:2], 1.0)
    slot_f = jnp.where(routed, r4[:, 0:1], -1.0)
    m2 = (slot_f == _iotaf((SB, NSLOT), 1)).astype(F32)
    y_add = jnp.dot(m2, y_ref[...], preferred_element_type=F32)
    out_ref[...] = sh_ref[...] * a_col + y_add


def _k2c(idx_row, p, shared_out, y_scaled):
    n = TOK // SB
    return pl.pallas_call(
        _k2c_body,
        grid=(n,),
        in_specs=[pl.BlockSpec((1, NMOE), lambda b: (0, 0)),
                  pl.BlockSpec((NMOE, 4), lambda b: (0, 0)),
                  pl.BlockSpec((SB, EMBED), lambda b: (b, 0)),
                  pl.BlockSpec((NSLOT, EMBED), lambda b: (0, 0))],
        out_specs=pl.BlockSpec((SB, EMBED), lambda b: (b, 0)),
        out_shape=jax.ShapeDtypeStruct((TOK, EMBED), F32),
        compiler_params=pltpu.CompilerParams(
            vmem_limit_bytes=100 * 1024 * 1024),
    )(idx_row, p, shared_out, y_scaled)


def _k3_body(bexp_ref, nab_ref, x_ref, w1_ref, b1_ref, w2_ref, b2_ref,
             g_ref, bb_ref, w3_ref, b3_ref, ew_ref, y_ref):
    b = pl.program_id(0)

    @pl.when(b < nab_ref[0])
    def _():
        h = _swiglu_ln(x_ref[...], w1_ref[0], b1_ref[0], w2_ref[0],
                       b2_ref[0], g_ref[...][None, :],
                       bb_ref[...][None, :])
        y_ref[...] = (_mm_t(h, w3_ref[0]) + b3_ref[0]) * ew_ref[...]

    @pl.when(b >= nab_ref[0])
    def _():
        y_ref[...] = jnp.zeros_like(y_ref)


def _k3(bexp, nabv, x_sorted, w1, b1, w2, b2, seln_g, seln_b, w3, b3,
        ew_sorted):
    grid_spec = pltpu.PrefetchScalarGridSpec(
        num_scalar_prefetch=2,
        grid=(NBLK,),
        in_specs=[
            pl.BlockSpec((BLK, EMBED), lambda b, be, na: (b, 0)),
            pl.BlockSpec((1, HE, EMBED), lambda b, be, na: (be[b], 0, 0)),
            pl.BlockSpec((1, 1, HE), lambda b, be, na: (be[b], 0, 0)),
            pl.BlockSpec((1, HE, EMBED), lambda b, be, na: (be[b], 0, 0)),
            pl.BlockSpec((1, 1, HE), lambda b, be, na: (be[b], 0, 0)),
            pl.BlockSpec((HE,), lambda b, be, na: (0,)),
            pl.BlockSpec((HE,), lambda b, be, na: (0,)),
            pl.BlockSpec((1, EMBED, HE), lambda b, be, na: (be[b], 0, 0)),
            pl.BlockSpec((1, 1, EMBED), lambda b, be, na: (be[b], 0, 0)),
            pl.BlockSpec((BLK, 1), lambda b, be, na: (b, 0)),
        ],
        out_specs=pl.BlockSpec((BLK, EMBED), lambda b, be, na: (b, 0)),
    )
    return pl.pallas_call(
        _k3_body,
        grid_spec=grid_spec,
        out_shape=jax.ShapeDtypeStruct((NSLOT, EMBED), F32),
        compiler_params=pltpu.CompilerParams(
            vmem_limit_bytes=100 * 1024 * 1024),
    )(bexp, nabv, x_sorted, w1, b1, w2, b2, seln_g, seln_b, w3, b3,
      ew_sorted)


def _sc_mesh():
    return plsc.VectorSubcoreMesh(core_axis_name="c", subcore_axis_name="s",
                                  num_cores=NC, num_subcores=NS)


def _wid():
    return lax.axis_index("s") * NC + lax.axis_index("c")


def _s1(gl_all, index_i):
    bpw = NMOE // NW

    @functools.partial(
        pl.kernel, mesh=_sc_mesh(),
        out_type=jax.ShapeDtypeStruct((NMOE, GL_W), F32),
        scratch_types=[pltpu.VMEM((bpw,), I32),
                       pltpu.VMEM((bpw, GL_W), F32),
                       pltpu.SemaphoreType.DMA])
    def k(gl_hbm, idx_hbm, out_hbm, idx_v, rows_v, sem):
        base = _wid() * bpw
        pltpu.sync_copy(idx_hbm.at[pl.ds(base, bpw)], idx_v)
        pltpu.async_copy(gl_hbm.at[idx_v], rows_v, sem).wait()
        pltpu.sync_copy(rows_v, out_hbm.at[pl.ds(base, bpw)])

    return k(gl_all, index_i)


def _s2(x, index_i, sorted_src):
    spw = NSLOT // NW
    ch = 16
    nb = 3
    gs = spw // nb

    @functools.partial(
        pl.kernel, mesh=_sc_mesh(),
        out_type=jax.ShapeDtypeStruct((NSLOT, EMBED), F32),
        compiler_params=pltpu.CompilerParams(needs_layout_passes=False),
        scratch_types=[pltpu.VMEM((NMOE,), I32),
                       pltpu.VMEM((spw,), I32),
                       pltpu.VMEM((spw,), I32)]
        + [pltpu.VMEM((gs, EMBED), F32)] * 3
        + [pltpu.SemaphoreType.DMA] * 6)
    def k(x_hbm, idx_hbm, src_hbm, out_hbm, tab_v, src_v, tid_v,
          b0, b1, b2, sg0, sg1, sg2, so0, so1, so2):
        base = _wid() * spw
        bufs = (b0, b1, b2)
        sg = (sg0, sg1, sg2)
        so = (so0, so1, so2)
        pltpu.sync_copy(idx_hbm, tab_v)
        pltpu.sync_copy(src_hbm.at[pl.ds(base, spw)], src_v)
        for c in range(spw // ch):
            sv = src_v[pl.ds(c * ch, ch)]
            tid_v[pl.ds(c * ch, ch)] = plsc.load_gather(tab_v, [sv])
        gh = [None] * nb
        oh = [None] * nb
        for s in range(nb):
            gh[s] = pltpu.async_copy(
                x_hbm.at[tid_v.at[pl.ds(s * gs, gs)]], bufs[s], sg[s])
        for s in range(nb):
            gh[s].wait()
            oh[s] = pltpu.async_copy(
                bufs[s], out_hbm.at[pl.ds(base + s * gs, gs)], so[s])
        for s in range(nb):
            oh[s].wait()

    return k(x, index_i, sorted_src)


def kernel(x, index, sw1_w, sw1_b, sw2_w, sw2_b, sln_g, sln_b, sw3_w,
           sw3_b, sg_w, sg_b, eg_w, eg_b, w1, b1, w2, b2, seln_g, seln_b,
           w3, b3, exp_bias):
    index_i = index.astype(I32)
    idx_f = index_i.astype(F32)
    idx_col = idx_f.reshape(NMOE, 1)
    idx_row = idx_f.reshape(1, NMOE)
    eb_row = exp_bias.reshape(1, E).astype(F32)
    gate_w = jnp.zeros((GL_W, EMBED), F32).at[0:1].set(sg_w)
    gate_w = gate_w.at[1:1 + E].set(eg_w)
    gate_b = jnp.zeros((GL_W,), F32).at[0:1].set(sg_b)
    gate_b = gate_b.at[1:1 + E].set(eg_b)

    gl_all = _k1g(x, gate_w, gate_b)
    gl_n = _s1(gl_all, index_i)
    dest_row, p, meta = _k2a(gl_n, idx_col, idx_row, eb_row)
    meta_flat = meta.reshape(128)
    bexp = meta_flat[32:56]
    nabv = meta_flat[58:59]
    sorted_src, ew_sorted = _k2b(dest_row, p, meta)
    x_sorted = _s2(x, index_i, sorted_src.reshape(NSLOT))
    shared_out = _k1s(x, sw1_w, sw1_b, sw2_w, sw2_b, sln_g, sln_b,
                      sw3_w, sw3_b)
    y_scaled = _k3(bexp, nabv, x_sorted, w1, b1.reshape(E, 1, HE), w2,
                   b2.reshape(E, 1, HE), seln_g, seln_b, w3,
                   b3.reshape(E, 1, EMBED), ew_sorted)
    out = _k2c(idx_row, p, shared_out, y_scaled)
    return out
